# baseline (speedup 1.0000x reference)
import functools

import jax
import jax.numpy as jnp
from jax import lax
from jax.experimental import pallas as pl
from jax.experimental.pallas import tpu as pltpu
from jax.experimental.pallas import tpu_sc as plsc

_NC = 2
_NS = 16
_NW = _NC * _NS

_IDX_W = 128
_K = 8
_CHUNK = _K * _IDX_W


def _gather_body(idx_hbm, table_hbm, out_hbm, idx_v, rows_v, gsem):
    b_total = out_hbm.shape[0]
    b_per_w = b_total // _NW
    n_chunks = b_per_w // _CHUNK

    wid = lax.axis_index("s") * _NC + lax.axis_index("c")
    row0 = wid * b_per_w

    def chunk_body(i, carry):
        base = pl.multiple_of(row0 + i * _CHUNK, _CHUNK)
        pltpu.sync_copy(idx_hbm.at[pl.ds(pl.multiple_of(base // _IDX_W, _K), _K)], idx_v)
        copies = []
        for j in range(_K):
            copies.append(
                pltpu.async_copy(
                    table_hbm.at[idx_v.at[j]],
                    rows_v.at[pl.ds(j * _IDX_W, _IDX_W)],
                    gsem,
                )
            )
        for cp in copies:
            cp.wait()
        pltpu.sync_copy(rows_v, out_hbm.at[pl.ds(base, _CHUNK)])
        return carry

    lax.fori_loop(0, n_chunks, chunk_body, 0)


@jax.jit
def _gather(idx2d, table):
    b_total = idx2d.shape[0] * _IDX_W
    d = table.shape[1]
    mesh = plsc.VectorSubcoreMesh(core_axis_name="c", subcore_axis_name="s")
    return pl.kernel(
        _gather_body,
        out_type=jax.ShapeDtypeStruct((b_total, d), jnp.float32),
        mesh=mesh,
        scratch_types=[
            pltpu.VMEM((_K, _IDX_W), jnp.int32),
            pltpu.VMEM((_CHUNK, d), jnp.float32),
            pltpu.SemaphoreType.DMA,
        ],
        compiler_params=pltpu.CompilerParams(use_tc_tiling_on_sc=False),
    )(idx2d, table)


def kernel(indices, table):
    b, s = indices.shape
    d = table.shape[1]
    idx2d = indices.reshape(-1, _IDX_W)
    out = _gather(idx2d, table)
    return out.reshape(b, s, d)


# speedup vs baseline: 1.0162x; 1.0162x over previous
import jax
import jax.numpy as jnp
from jax import lax
from jax.experimental import pallas as pl
from jax.experimental.pallas import tpu as pltpu
from jax.experimental.pallas import tpu_sc as plsc

_NC = 2
_NS = 16
_NW = _NC * _NS

_IDX_W = 128
_K = 4
_CHUNK = _K * _IDX_W


def _gather_body(idx_hbm, table_hbm, out_hbm, idx_v, rows_a, rows_b,
                 gsem, oa_sem, ob_sem):
    d = table_hbm.shape[1]
    b_total = out_hbm.shape[0]
    b_per_w = b_total // _NW
    n_chunks = b_per_w // _CHUNK
    n_pairs = n_chunks // 2
    idx_rows = b_per_w // _IDX_W

    wid = lax.axis_index("s") * _NC + lax.axis_index("c")
    row0 = pl.multiple_of(wid * b_per_w, _CHUNK)
    irow0 = pl.multiple_of(wid * idx_rows, 8)

    pltpu.sync_copy(idx_hbm.at[pl.ds(irow0, idx_rows)], idx_v)

    def out_slice(c):
        return out_hbm.at[pl.ds(pl.multiple_of(row0 + c * _CHUNK, _CHUNK), _CHUNK)]

    def run_chunk(c, buf, osem):
        copies = []
        for j in range(_K):
            copies.append(
                pltpu.async_copy(
                    table_hbm.at[idx_v.at[c * _K + j]],
                    buf.at[pl.ds(j * _IDX_W, _IDX_W)],
                    gsem,
                )
            )
        for cp in copies:
            cp.wait()
        pltpu.async_copy(buf, out_slice(c), osem)

    def pair_body(i, carry):
        ca = 2 * i
        cb = 2 * i + 1

        @pl.when(i > 0)
        def _():
            pltpu.make_async_copy(rows_a, out_slice(ca), oa_sem).wait()

        run_chunk(ca, rows_a, oa_sem)

        @pl.when(i > 0)
        def _():
            pltpu.make_async_copy(rows_b, out_slice(cb), ob_sem).wait()

        run_chunk(cb, rows_b, ob_sem)
        return carry

    lax.fori_loop(0, n_pairs, pair_body, 0)

    last = n_chunks - 1
    pltpu.make_async_copy(rows_a, out_slice(last), oa_sem).wait()
    pltpu.make_async_copy(rows_b, out_slice(last), ob_sem).wait()


@jax.jit
def _gather(idx2d, table):
    b_total = idx2d.shape[0] * _IDX_W
    d = table.shape[1]
    b_per_w = b_total // _NW
    mesh = plsc.VectorSubcoreMesh(core_axis_name="c", subcore_axis_name="s")
    return pl.kernel(
        _gather_body,
        out_type=jax.ShapeDtypeStruct((b_total, d), jnp.float32),
        mesh=mesh,
        scratch_types=[
            pltpu.VMEM((b_per_w // _IDX_W, _IDX_W), jnp.int32),
            pltpu.VMEM((_CHUNK, d), jnp.float32),
            pltpu.VMEM((_CHUNK, d), jnp.float32),
            pltpu.SemaphoreType.DMA,
            pltpu.SemaphoreType.DMA,
            pltpu.SemaphoreType.DMA,
        ],
        compiler_params=pltpu.CompilerParams(use_tc_tiling_on_sc=False),
    )(idx2d, table)


def kernel(indices, table):
    b, s = indices.shape
    d = table.shape[1]
    idx2d = indices.reshape(-1, _IDX_W)
    out = _gather(idx2d, table)
    return out.reshape(b, s, d)


# speedup vs baseline: 1.0415x; 1.0249x over previous
import jax
import jax.numpy as jnp
from jax import lax
from jax.experimental import pallas as pl
from jax.experimental.pallas import tpu as pltpu
from jax.experimental.pallas import tpu_sc as plsc

_NC = 2
_NS = 16
_NW = _NC * _NS

_IDX_W = 128
_K = 4
_CHUNK = _K * _IDX_W


def _gather_body(idx_hbm, table_hbm, out_hbm, idx_v, rows_a, rows_b,
                 gsem, oa_sem, ob_sem):
    d = table_hbm.shape[1]
    b_total = out_hbm.shape[0]
    b_per_w = b_total // _NW
    n_chunks = b_per_w // _CHUNK
    n_pairs = n_chunks // 2
    idx_rows = b_per_w // _IDX_W

    wid = lax.axis_index("s") * _NC + lax.axis_index("c")
    row0 = pl.multiple_of(wid * b_per_w, _CHUNK)
    irow0 = pl.multiple_of(wid * idx_rows, 8)

    pltpu.sync_copy(idx_hbm.at[pl.ds(irow0, idx_rows)], idx_v)

    def out_slice(c):
        return out_hbm.at[pl.ds(pl.multiple_of(row0 + c * _CHUNK, _CHUNK), _CHUNK)]

    def run_chunk(c, buf, osem):
        copies = []
        for j in range(_K):
            copies.append(
                pltpu.async_copy(
                    table_hbm.at[idx_v.at[c * _K + j]],
                    buf.at[pl.ds(j * _IDX_W, _IDX_W)],
                    gsem,
                )
            )
        for cp in copies:
            cp.wait()
        pltpu.async_copy(buf, out_slice(c), osem)

    def pair_body(i, carry):
        ca = 2 * i
        cb = 2 * i + 1

        @pl.when(i > 0)
        def _():
            pltpu.make_async_copy(rows_a, out_slice(ca), oa_sem).wait()

        run_chunk(ca, rows_a, oa_sem)

        @pl.when(i > 0)
        def _():
            pltpu.make_async_copy(rows_b, out_slice(cb), ob_sem).wait()

        run_chunk(cb, rows_b, ob_sem)
        return carry

    lax.fori_loop(0, n_pairs, pair_body, 0)

    last = n_chunks - 1
    pltpu.make_async_copy(rows_a, out_slice(last), oa_sem).wait()
    pltpu.make_async_copy(rows_b, out_slice(last), ob_sem).wait()


@jax.jit
def _gather(idx2d, table):
    b_total = idx2d.shape[0] * _IDX_W
    d = table.shape[1]
    b_per_w = b_total // _NW
    mesh = plsc.VectorSubcoreMesh(core_axis_name="c", subcore_axis_name="s")
    return pl.kernel(
        _gather_body,
        out_type=jax.ShapeDtypeStruct((b_total, d), jnp.float32),
        mesh=mesh,
        scratch_types=[
            pltpu.VMEM((b_per_w // _IDX_W, _IDX_W), jnp.int32),
            pltpu.VMEM((_CHUNK, d), jnp.float32),
            pltpu.VMEM((_CHUNK, d), jnp.float32),
            pltpu.SemaphoreType.DMA,
            pltpu.SemaphoreType.DMA,
            pltpu.SemaphoreType.DMA,
        ],
        compiler_params=pltpu.CompilerParams(use_tc_tiling_on_sc=False),
    )(idx2d, table)


def kernel(indices, table):
    b, s = indices.shape
    d = table.shape[1]
    idx2d = indices.T.reshape(-1, _IDX_W)
    out = _gather(idx2d, table)
    return out.reshape(s, b, d).transpose(1, 0, 2)
